# baseline (speedup 1.0000x reference)
import jax
import jax.numpy as jnp
from jax.experimental import pallas as pl

NBITS = 256
HIDDEN = 256
SPATIAL = 32
B = 128
C = 128
HW = SPATIAL * SPATIAL


def _aux_body(msg_ref, we_ref, wo_ref, aux_ref):
    we = we_ref[...]
    d = wo_ref[...] - we
    base = jnp.sum(we, axis=0, keepdims=True)
    aux_ref[...] = jax.lax.dot_general(
        msg_ref[...], d, (((1,), (0,)), ((), ())),
        preferred_element_type=jnp.float32) + base


def _bcast_body(aux_ref, lat_ref, out_ref):
    out_ref[0, :C, :] = lat_ref[0]
    out_ref[0, C:, :] = jnp.broadcast_to(aux_ref[0], (HIDDEN, HW))


def kernel(latents, msg, W):
    lat3 = latents.reshape(B, C, HW)
    msg_f = msg.astype(jnp.float32)
    we = W[0::2]
    wo = W[1::2]
    aux = pl.pallas_call(
        _aux_body,
        in_specs=[
            pl.BlockSpec((B, NBITS), lambda: (0, 0)),
            pl.BlockSpec((NBITS, HIDDEN), lambda: (0, 0)),
            pl.BlockSpec((NBITS, HIDDEN), lambda: (0, 0)),
        ],
        out_specs=pl.BlockSpec((B, HIDDEN), lambda: (0, 0)),
        out_shape=jax.ShapeDtypeStruct((B, HIDDEN), jnp.float32),
    )(msg_f, we, wo)
    aux3 = aux.reshape(B, HIDDEN, 1)
    out = pl.pallas_call(
        _bcast_body,
        grid=(B,),
        in_specs=[
            pl.BlockSpec((1, HIDDEN, 1), lambda b: (b, 0, 0)),
            pl.BlockSpec((1, C, HW), lambda b: (b, 0, 0)),
        ],
        out_specs=pl.BlockSpec((1, C + HIDDEN, HW), lambda b: (b, 0, 0)),
        out_shape=jax.ShapeDtypeStruct((B, C + HIDDEN, HW), jnp.float32),
    )(aux3, lat3)
    return out.reshape(B, C + HIDDEN, SPATIAL, SPATIAL)
